# baseline (speedup 1.0000x reference)
import functools
import jax
import jax.numpy as jnp
from jax import lax
from jax.experimental import pallas as pl
from jax.experimental.pallas import tpu as pltpu
from jax.experimental.pallas import tpu_sc as plsc

B, NP, NA, DIN, HID, H = 256, 16, 16, 256, 128, 4
N = B * (NP + NA)
E = 65536
NC, NS, L = 2, 16, 16
NW = NC * NS
DP = 144

_HI = jax.lax.Precision.DEFAULT


_BM = 512


def _pad_cols(bm):
    return jnp.concatenate(
        [jnp.ones((bm, 1), jnp.float32), jnp.zeros((bm, DP - HID - 1),
                                                   jnp.float32)], axis=1)


def _proj_body(feat_ref, w_ref, wa_ref, hh_ref, es_ref, ed_ref):
    h = pl.program_id(1)
    f = feat_ref[...]
    blk = jnp.dot(f, w_ref[0], preferred_element_type=jnp.float32,
                  precision=_HI)
    hh_ref[0] = jnp.concatenate([blk, _pad_cols(f.shape[0])], axis=1)

    @pl.when(h == 0)
    def _():
        es_ref[...] = jnp.dot(f, wa_ref[0, 0],
                              preferred_element_type=jnp.float32,
                              precision=_HI)
        ed_ref[...] = jnp.dot(f, wa_ref[0, 1],
                              preferred_element_type=jnp.float32,
                              precision=_HI)


def _tc_proj(feat, Ws, Wa):
    K = feat.shape[1]
    ni = N // _BM
    return pl.pallas_call(
        _proj_body,
        grid=(ni, H),
        in_specs=[
            pl.BlockSpec((_BM, K), lambda i, h: (i, 0)),
            pl.BlockSpec((1, K, HID), lambda i, h: (i // (ni // 2), 0, h)),
            pl.BlockSpec((1, 2, K, 16), lambda i, h: (i // (ni // 2), 0, 0, 0)),
        ],
        out_specs=[
            pl.BlockSpec((1, _BM, DP), lambda i, h: (h, i, 0)),
            pl.BlockSpec((_BM, 16), lambda i, h: (i, 0)),
            pl.BlockSpec((_BM, 16), lambda i, h: (i, 0)),
        ],
        out_shape=[
            jax.ShapeDtypeStruct((H, N, DP), jnp.float32),
            jax.ShapeDtypeStruct((N, 16), jnp.float32),
            jax.ShapeDtypeStruct((N, 16), jnp.float32),
        ],
    )(feat, Ws, Wa)


def _elu(x):
    return jnp.where(x > 0, x, jnp.exp(jnp.minimum(x, 0.0)) - 1.0)


def _h1_from(out_ref):
    cols = []
    for h in range(H):
        s = out_ref[h, :, HID:HID + 1] + 1e-9
        cols.append(_elu(out_ref[h, :, :HID] / s))
    return jnp.concatenate(cols, axis=1)


def _norm_proj_body(out_ref, w_ref, wa_ref, h1_ref, hh_ref, es_ref, ed_ref):
    h = pl.program_id(1)
    h1 = _h1_from(out_ref)
    blk = jnp.dot(h1, w_ref[0], preferred_element_type=jnp.float32,
                  precision=_HI)
    hh_ref[0] = jnp.concatenate([blk, _pad_cols(h1.shape[0])], axis=1)

    @pl.when(h == 0)
    def _():
        h1_ref[...] = h1
        es_ref[...] = jnp.dot(h1, wa_ref[0, 0],
                              preferred_element_type=jnp.float32,
                              precision=_HI)
        ed_ref[...] = jnp.dot(h1, wa_ref[0, 1],
                              preferred_element_type=jnp.float32,
                              precision=_HI)


def _tc_norm_proj(out_unnorm, Ws, Wa):
    ni = N // _BM
    return pl.pallas_call(
        _norm_proj_body,
        grid=(ni, H),
        in_specs=[
            pl.BlockSpec((H, _BM, DP), lambda i, h: (0, i, 0)),
            pl.BlockSpec((1, H * HID, HID),
                         lambda i, h: (i // (ni // 2), 0, h)),
            pl.BlockSpec((1, 2, H * HID, 16),
                         lambda i, h: (i // (ni // 2), 0, 0, 0)),
        ],
        out_specs=[
            pl.BlockSpec((_BM, H * HID), lambda i, h: (i, 0)),
            pl.BlockSpec((1, _BM, DP), lambda i, h: (h, i, 0)),
            pl.BlockSpec((_BM, 16), lambda i, h: (i, 0)),
            pl.BlockSpec((_BM, 16), lambda i, h: (i, 0)),
        ],
        out_shape=[
            jax.ShapeDtypeStruct((N, H * HID), jnp.float32),
            jax.ShapeDtypeStruct((H, N, DP), jnp.float32),
            jax.ShapeDtypeStruct((N, 16), jnp.float32),
            jax.ShapeDtypeStruct((N, 16), jnp.float32),
        ],
    )(out_unnorm, Ws, Wa)


def _state1_body(h1_ref, obs_ref, wself_ref, wnode_ref, st_ref):
    h1 = h1_ref[...]
    nm = (h1[:B * NP].reshape(B, NP, H * HID).sum(axis=1)
          + h1[B * NP:].reshape(B, NA, H * HID).sum(axis=1)) * (1.0 / (NP + NA))
    st = (jnp.dot(obs_ref[...], wself_ref[...],
                  preferred_element_type=jnp.float32, precision=_HI)
          + jnp.dot(nm, wnode_ref[...],
                    preferred_element_type=jnp.float32, precision=_HI))
    st_ref[...] = _elu(st)


def _tc_state1(h1, cent_obs, W_self, W_node):
    return pl.pallas_call(
        _state1_body,
        out_shape=jax.ShapeDtypeStruct((B, H * HID), jnp.float32),
    )(h1, cent_obs, W_self, W_node)


def _final_body(out_ref, st1_ref, ws2_ref, wns2_ref, x_ref, state_ref):
    q = jnp.zeros((N, HID), jnp.float32)
    for h in range(H):
        s = out_ref[h, :, HID:HID + 1] + 1e-9
        q = q + out_ref[h, :, :HID] / s
    q = q * (1.0 / H)
    x = jnp.concatenate([q[:B * NP].reshape(B, NP, HID),
                         q[B * NP:].reshape(B, NA, HID)], axis=1)
    x_ref[...] = x
    nm2 = x.sum(axis=1) * (1.0 / (NP + NA))
    state_ref[...] = (
        jnp.dot(st1_ref[...], ws2_ref[...],
                preferred_element_type=jnp.float32, precision=_HI)
        + jnp.dot(nm2, wns2_ref[...],
                  preferred_element_type=jnp.float32, precision=_HI))


def _tc_final(out_unnorm, state1, W_s2, W_ns2):
    return pl.pallas_call(
        _final_body,
        out_shape=[
            jax.ShapeDtypeStruct((B, NP + NA, HID), jnp.float32),
            jax.ShapeDtypeStruct((B, W_s2.shape[1]), jnp.float32),
        ],
    )(out_unnorm, state1, W_s2, W_ns2)


_CH = 128
_EPW = E // NW
_NCH1 = _EPW // _CH
_RPT = N // NS
_EPT2 = E // NS
_NCH2 = _EPT2 // _CH

_sc_mesh = plsc.VectorSubcoreMesh(core_axis_name="c", subcore_axis_name="s")
_sc_params = pltpu.CompilerParams(use_tc_tiling_on_sc=False)


def _sc_pass1_body(es_hbm, ed_hbm, src_hbm, dst_hbm, ex_hbm,
                   src_all, dst_all, srows0, drows0, srows1, drows1,
                   exfull, gs0, gs1):
    c = lax.axis_index("c")
    sid = lax.axis_index("s")
    wid = sid * NC + c
    base_w = wid * _EPW
    pltpu.sync_copy(src_hbm.at[pl.ds(base_w, _EPW)], src_all)
    pltpu.sync_copy(dst_hbm.at[pl.ds(base_w, _EPW)], dst_all)

    def _gather(k, srows, drows, sem):
        pltpu.async_copy(es_hbm.at[src_all.at[pl.ds(k * _CH, _CH)]],
                         srows, sem)
        pltpu.async_copy(ed_hbm.at[dst_all.at[pl.ds(k * _CH, _CH)]],
                         drows, sem)

    def _gwait(k, srows, drows, sem):
        pltpu.make_async_copy(es_hbm.at[src_all.at[pl.ds(k * _CH, _CH)]],
                              srows, sem).wait()
        pltpu.make_async_copy(ed_hbm.at[dst_all.at[pl.ds(k * _CH, _CH)]],
                              drows, sem).wait()

    def _compute(k, srows, drows):
        @plsc.parallel_loop(0, _CH, unroll=4)
        def _edge(e):
            z = srows[e, :] + drows[e, :]
            z = jnp.where(z >= 0.0, z, z * jnp.float32(0.2))
            exfull[k * _CH + e, :] = jnp.exp(z)

    _gather(0, srows0, drows0, gs0)

    def _pair(j, _):
        k0 = 2 * j
        _gwait(k0, srows0, drows0, gs0)
        _gather(k0 + 1, srows1, drows1, gs1)
        _compute(k0, srows0, drows0)
        _gwait(k0 + 1, srows1, drows1, gs1)

        @pl.when(j < _NCH1 // 2 - 1)
        def _():
            _gather(k0 + 2, srows0, drows0, gs0)
        _compute(k0 + 1, srows1, drows1)
        return 0

    lax.fori_loop(0, _NCH1 // 2, _pair, 0)
    pltpu.sync_copy(exfull, ex_hbm.at[pl.ds(base_w, _EPW)])


def _sc_pass1(es_tbl, ed_tbl, src, dst):
    k = pl.kernel(
        _sc_pass1_body,
        out_type=jax.ShapeDtypeStruct((E, 16), jnp.float32),
        mesh=_sc_mesh,
        scratch_types=[
            pltpu.VMEM((_EPW,), jnp.int32),
            pltpu.VMEM((_EPW,), jnp.int32),
            pltpu.VMEM((_CH, 16), jnp.float32),
            pltpu.VMEM((_CH, 16), jnp.float32),
            pltpu.VMEM((_CH, 16), jnp.float32),
            pltpu.VMEM((_CH, 16), jnp.float32),
            pltpu.VMEM((_EPW, 16), jnp.float32),
            pltpu.SemaphoreType.DMA,
            pltpu.SemaphoreType.DMA,
        ],
        compiler_params=_sc_params,
    )
    return k(es_tbl, ed_tbl, src, dst)


def _scale_rows(rows, exch, head):
    @plsc.parallel_loop(0, _CH, unroll=8)
    def _edge(e):
        exv = exch[e, :][head]
        for cc in range(DP // L):
            v = rows[e, pl.ds(cc * L, L)]
            rows[e, pl.ds(cc * L, L)] = v * exv


def _sc_pass2_head(head, sid, hh_hbm, ex_hbm, out_hbm, src_all, dst2d,
                   exch0, exch1, rows0, rows1, zb, acc_sh, gs0, gs1, ss0, ss1):
    def _z(i, _):
        for cc in range(DP // L):
            zb[i, pl.ds(cc * L, L)] = jnp.zeros((L,), jnp.float32)
        return 0
    lax.fori_loop(0, 32, _z, 0)
    for j in range(_RPT // 32):
        pltpu.sync_copy(zb, acc_sh.at[pl.ds(sid * _RPT + j * 32, 32)])
    plsc.subcore_barrier()

    tbl = hh_hbm.at[head]
    base_t = sid * _EPT2

    def _gather(k, rows, exch, sem):
        pltpu.async_copy(ex_hbm.at[pl.ds(base_t + k * _CH, _CH)], exch, sem)
        pltpu.async_copy(tbl.at[src_all.at[pl.ds(k * _CH, _CH)]], rows, sem)

    def _gwait(k, rows, exch, sem):
        pltpu.make_async_copy(ex_hbm.at[pl.ds(base_t + k * _CH, _CH)],
                              exch, sem).wait()
        pltpu.make_async_copy(tbl.at[src_all.at[pl.ds(k * _CH, _CH)]],
                              rows, sem).wait()

    def _scat(k, rows, sem):
        pltpu.async_copy(rows, acc_sh.at[dst2d.at[k]], sem, add=True)

    def _swait(k, rows, sem):
        pltpu.make_async_copy(rows, acc_sh.at[dst2d.at[k]], sem).wait()

    _gather(0, rows0, exch0, gs0)

    def _pair(j, _):
        k0 = 2 * j
        _gwait(k0, rows0, exch0, gs0)

        @pl.when(j > 0)
        def _():
            _swait(k0, rows1, ss1)
        _gather(k0 + 1, rows1, exch1, gs1)
        _scale_rows(rows0, exch0, head)
        _scat(k0, rows0, ss0)
        _gwait(k0 + 1, rows1, exch1, gs1)
        _swait(k0, rows0, ss0)

        @pl.when(j < _NCH2 // 2 - 1)
        def _():
            _gather(k0 + 2, rows0, exch0, gs0)
        _scale_rows(rows1, exch1, head)
        _scat(k0 + 1, rows1, ss1)
        return 0

    lax.fori_loop(0, _NCH2 // 2, _pair, 0)
    _swait(_NCH2 - 1, rows1, ss1)
    plsc.subcore_barrier()
    pltpu.sync_copy(acc_sh.at[pl.ds(sid * _RPT, _RPT)],
                    out_hbm.at[head, pl.ds(sid * _RPT, _RPT)])
    plsc.subcore_barrier()


def _sc_pass2_body(hh_hbm, src_hbm, dst_hbm, ex_hbm, out_hbm,
                   src_all, dst2d, exch0, exch1, rows0, rows1, zb, acc_sh,
                   gs0, gs1, ss0, ss1):
    c = lax.axis_index("c")
    sid = lax.axis_index("s")
    base_t = sid * _EPT2
    pltpu.sync_copy(src_hbm.at[pl.ds(base_t, _EPT2)], src_all)
    pltpu.sync_copy(dst_hbm.at[pl.ds(sid * _NCH2, _NCH2)], dst2d)
    for core in range(NC):
        @pl.when(c == core)
        def _():
            for sub in range(2):
                _sc_pass2_head(2 * core + sub, sid, hh_hbm, ex_hbm, out_hbm,
                               src_all, dst2d, exch0, exch1, rows0, rows1,
                               zb, acc_sh, gs0, gs1, ss0, ss1)


def _sc_pass2(h_heads, src, dst2d, ex):
    k = pl.kernel(
        _sc_pass2_body,
        out_type=jax.ShapeDtypeStruct((H, N, DP), jnp.float32),
        mesh=_sc_mesh,
        scratch_types=[
            pltpu.VMEM((_EPT2,), jnp.int32),
            pltpu.VMEM((_NCH2, _CH), jnp.int32),
            pltpu.VMEM((_CH, 16), jnp.float32),
            pltpu.VMEM((_CH, 16), jnp.float32),
            pltpu.VMEM((_CH, DP), jnp.float32),
            pltpu.VMEM((_CH, DP), jnp.float32),
            pltpu.VMEM((32, DP), jnp.float32),
            pltpu.VMEM_SHARED((N, DP), jnp.float32),
            pltpu.SemaphoreType.DMA,
            pltpu.SemaphoreType.DMA,
            pltpu.SemaphoreType.DMA,
            pltpu.SemaphoreType.DMA,
        ],
        compiler_params=_sc_params,
    )
    return k(h_heads, src, dst2d, ex)


def _fold_att(W, a):
    S = jnp.zeros((H * HID, 16), jnp.float32)
    for h in range(H):
        S = S.at[h * HID:(h + 1) * HID, h].set(a[h])
    return W @ S


def kernel(feat_P, feat_A, cent_obs, edge_index, batch_size,
           W_P1, W_A1, a_src1, a_dst1, W_s1_self, W_s1_node,
           W_P2, W_A2, a_src2, a_dst2, W_s2, W_ns2):
    src = edge_index[0].astype(jnp.int32)
    dst = edge_index[1].astype(jnp.int32)
    dst2d = dst.reshape(E // _CH, _CH)
    feat = jnp.concatenate([feat_P, feat_A], axis=0)

    W1s = jnp.stack([W_P1, W_A1])
    Wa1 = jnp.stack([
        jnp.stack([_fold_att(W_P1, a_src1), _fold_att(W_P1, a_dst1)]),
        jnp.stack([_fold_att(W_A1, a_src1), _fold_att(W_A1, a_dst1)])])
    W2s = jnp.stack([W_P2, W_A2])
    Wa2 = jnp.stack([
        jnp.stack([_fold_att(W_P2, a_src2), _fold_att(W_P2, a_dst2)]),
        jnp.stack([_fold_att(W_A2, a_src2), _fold_att(W_A2, a_dst2)])])

    h1_heads, es1, ed1 = _tc_proj(feat, W1s, Wa1)
    ex1 = _sc_pass1(es1, ed1, src, dst)
    out1 = _sc_pass2(h1_heads, src, dst2d, ex1)

    h1, h2_heads, es2, ed2 = _tc_norm_proj(out1, W2s, Wa2)
    state1 = _tc_state1(h1, cent_obs, W_s1_self, W_s1_node)
    ex2 = _sc_pass1(es2, ed2, src, dst)
    out2 = _sc_pass2(h2_heads, src, dst2d, ex2)

    x, state = _tc_final(out2, state1, W_s2, W_ns2)
    return (x, state)


# speedup vs baseline: 1.0069x; 1.0069x over previous
import functools
import jax
import jax.numpy as jnp
from jax import lax
from jax.experimental import pallas as pl
from jax.experimental.pallas import tpu as pltpu
from jax.experimental.pallas import tpu_sc as plsc

B, NP, NA, DIN, HID, H = 256, 16, 16, 256, 128, 4
N = B * (NP + NA)
E = 65536
NC, NS, L = 2, 16, 16
NW = NC * NS
DP = 144

_HI = jax.lax.Precision.DEFAULT


_BM = 512


def _pad_cols(bm):
    return jnp.concatenate(
        [jnp.ones((bm, 1), jnp.float32), jnp.zeros((bm, DP - HID - 1),
                                                   jnp.float32)], axis=1)


def _proj_body(feat_ref, w_ref, wa_ref, hh_ref, es_ref, ed_ref):
    h = pl.program_id(1)
    f = feat_ref[...]
    blk = jnp.dot(f, w_ref[0], preferred_element_type=jnp.float32,
                  precision=_HI)
    hh_ref[0] = jnp.concatenate([blk, _pad_cols(f.shape[0])], axis=1)

    @pl.when(h == 0)
    def _():
        es_ref[...] = jnp.dot(f, wa_ref[0, 0],
                              preferred_element_type=jnp.float32,
                              precision=_HI)
        ed_ref[...] = jnp.dot(f, wa_ref[0, 1],
                              preferred_element_type=jnp.float32,
                              precision=_HI)


def _tc_proj(feat, Ws, Wa):
    K = feat.shape[1]
    ni = N // _BM
    return pl.pallas_call(
        _proj_body,
        grid=(ni, H),
        in_specs=[
            pl.BlockSpec((_BM, K), lambda i, h: (i, 0)),
            pl.BlockSpec((1, K, HID), lambda i, h: (i // (ni // 2), 0, h)),
            pl.BlockSpec((1, 2, K, 16), lambda i, h: (i // (ni // 2), 0, 0, 0)),
        ],
        out_specs=[
            pl.BlockSpec((1, _BM, DP), lambda i, h: (h, i, 0)),
            pl.BlockSpec((_BM, 16), lambda i, h: (i, 0)),
            pl.BlockSpec((_BM, 16), lambda i, h: (i, 0)),
        ],
        out_shape=[
            jax.ShapeDtypeStruct((H, N, DP), jnp.float32),
            jax.ShapeDtypeStruct((N, 16), jnp.float32),
            jax.ShapeDtypeStruct((N, 16), jnp.float32),
        ],
    )(feat, Ws, Wa)


def _elu(x):
    return jnp.where(x > 0, x, jnp.exp(jnp.minimum(x, 0.0)) - 1.0)


def _h1_from(out_ref):
    cols = []
    for h in range(H):
        s = out_ref[h, :, HID:HID + 1] + 1e-9
        cols.append(_elu(out_ref[h, :, :HID] / s))
    return jnp.concatenate(cols, axis=1)


def _norm_proj_body(out_ref, w_ref, wa_ref, h1sum_ref, hh_ref, es_ref,
                    ed_ref):
    i = pl.program_id(0)
    h = pl.program_id(1)
    h1 = _h1_from(out_ref)
    blk = jnp.dot(h1, w_ref[0], preferred_element_type=jnp.float32,
                  precision=_HI)
    hh_ref[0] = jnp.concatenate([blk, _pad_cols(h1.shape[0])], axis=1)

    @pl.when(h == 0)
    def _():
        es_ref[...] = jnp.dot(h1, wa_ref[0, 0],
                              preferred_element_type=jnp.float32,
                              precision=_HI)
        ed_ref[...] = jnp.dot(h1, wa_ref[0, 1],
                              preferred_element_type=jnp.float32,
                              precision=_HI)

        @pl.when(i == 0)
        def _():
            h1sum_ref[...] = jnp.zeros((B, H * HID), jnp.float32)
        ng = _BM // NP
        g0 = (i % (B // ng)) * ng
        h1sum_ref[pl.ds(g0, ng), :] = (
            h1sum_ref[pl.ds(g0, ng), :]
            + h1.reshape(ng, NP, H * HID).sum(axis=1))


def _tc_norm_proj(out_unnorm, Ws, Wa):
    ni = N // _BM
    return pl.pallas_call(
        _norm_proj_body,
        grid=(ni, H),
        in_specs=[
            pl.BlockSpec((H, _BM, DP), lambda i, h: (0, i, 0)),
            pl.BlockSpec((1, H * HID, HID),
                         lambda i, h: (i // (ni // 2), 0, h)),
            pl.BlockSpec((1, 2, H * HID, 16),
                         lambda i, h: (i // (ni // 2), 0, 0, 0)),
        ],
        out_specs=[
            pl.BlockSpec((B, H * HID), lambda i, h: (0, 0)),
            pl.BlockSpec((1, _BM, DP), lambda i, h: (h, i, 0)),
            pl.BlockSpec((_BM, 16), lambda i, h: (i, 0)),
            pl.BlockSpec((_BM, 16), lambda i, h: (i, 0)),
        ],
        out_shape=[
            jax.ShapeDtypeStruct((B, H * HID), jnp.float32),
            jax.ShapeDtypeStruct((H, N, DP), jnp.float32),
            jax.ShapeDtypeStruct((N, 16), jnp.float32),
            jax.ShapeDtypeStruct((N, 16), jnp.float32),
        ],
    )(out_unnorm, Ws, Wa)


def _final_body(out_ref, h1sum_ref, obs_ref, wself_ref, wnode_ref,
                ws2_ref, wns2_ref, x_ref, state_ref):
    st1 = _elu(jnp.dot(obs_ref[...], wself_ref[...],
                       preferred_element_type=jnp.float32, precision=_HI)
               + jnp.dot(h1sum_ref[...] * (1.0 / (NP + NA)), wnode_ref[...],
                         preferred_element_type=jnp.float32, precision=_HI))
    q = jnp.zeros((N, HID), jnp.float32)
    for h in range(H):
        s = out_ref[h, :, HID:HID + 1] + 1e-9
        q = q + out_ref[h, :, :HID] / s
    q = q * (1.0 / H)
    x = jnp.concatenate([q[:B * NP].reshape(B, NP, HID),
                         q[B * NP:].reshape(B, NA, HID)], axis=1)
    x_ref[...] = x
    nm2 = x.sum(axis=1) * (1.0 / (NP + NA))
    state_ref[...] = (
        jnp.dot(st1, ws2_ref[...],
                preferred_element_type=jnp.float32, precision=_HI)
        + jnp.dot(nm2, wns2_ref[...],
                  preferred_element_type=jnp.float32, precision=_HI))


def _tc_final(out_unnorm, h1sum, cent_obs, W_self, W_node, W_s2, W_ns2):
    return pl.pallas_call(
        _final_body,
        out_shape=[
            jax.ShapeDtypeStruct((B, NP + NA, HID), jnp.float32),
            jax.ShapeDtypeStruct((B, W_s2.shape[1]), jnp.float32),
        ],
    )(out_unnorm, h1sum, cent_obs, W_self, W_node, W_s2, W_ns2)


_CH = 128
_EPW = E // NW
_NCH1 = _EPW // _CH
_RPT = N // NS
_EPT2 = E // NS
_NCH2 = _EPT2 // _CH

_sc_mesh = plsc.VectorSubcoreMesh(core_axis_name="c", subcore_axis_name="s")
_sc_params = pltpu.CompilerParams(use_tc_tiling_on_sc=False)


def _sc_pass1_body(es_hbm, ed_hbm, src_hbm, dst_hbm, ex_hbm,
                   src_all, dst_all, srows0, drows0, srows1, drows1,
                   exfull, gs0, gs1):
    c = lax.axis_index("c")
    sid = lax.axis_index("s")
    wid = sid * NC + c
    base_w = wid * _EPW
    pltpu.sync_copy(src_hbm.at[pl.ds(base_w, _EPW)], src_all)
    pltpu.sync_copy(dst_hbm.at[pl.ds(base_w, _EPW)], dst_all)

    def _gather(k, srows, drows, sem):
        pltpu.async_copy(es_hbm.at[src_all.at[pl.ds(k * _CH, _CH)]],
                         srows, sem)
        pltpu.async_copy(ed_hbm.at[dst_all.at[pl.ds(k * _CH, _CH)]],
                         drows, sem)

    def _gwait(k, srows, drows, sem):
        pltpu.make_async_copy(es_hbm.at[src_all.at[pl.ds(k * _CH, _CH)]],
                              srows, sem).wait()
        pltpu.make_async_copy(ed_hbm.at[dst_all.at[pl.ds(k * _CH, _CH)]],
                              drows, sem).wait()

    def _compute(k, srows, drows):
        @plsc.parallel_loop(0, _CH, unroll=4)
        def _edge(e):
            z = srows[e, :] + drows[e, :]
            z = jnp.where(z >= 0.0, z, z * jnp.float32(0.2))
            exfull[k * _CH + e, :] = jnp.exp(z)

    _gather(0, srows0, drows0, gs0)

    def _pair(j, _):
        k0 = 2 * j
        _gwait(k0, srows0, drows0, gs0)
        _gather(k0 + 1, srows1, drows1, gs1)
        _compute(k0, srows0, drows0)
        _gwait(k0 + 1, srows1, drows1, gs1)

        @pl.when(j < _NCH1 // 2 - 1)
        def _():
            _gather(k0 + 2, srows0, drows0, gs0)
        _compute(k0 + 1, srows1, drows1)
        return 0

    lax.fori_loop(0, _NCH1 // 2, _pair, 0)
    pltpu.sync_copy(exfull, ex_hbm.at[pl.ds(base_w, _EPW)])


def _sc_pass1(es_tbl, ed_tbl, src, dst):
    k = pl.kernel(
        _sc_pass1_body,
        out_type=jax.ShapeDtypeStruct((E, 16), jnp.float32),
        mesh=_sc_mesh,
        scratch_types=[
            pltpu.VMEM((_EPW,), jnp.int32),
            pltpu.VMEM((_EPW,), jnp.int32),
            pltpu.VMEM((_CH, 16), jnp.float32),
            pltpu.VMEM((_CH, 16), jnp.float32),
            pltpu.VMEM((_CH, 16), jnp.float32),
            pltpu.VMEM((_CH, 16), jnp.float32),
            pltpu.VMEM((_EPW, 16), jnp.float32),
            pltpu.SemaphoreType.DMA,
            pltpu.SemaphoreType.DMA,
        ],
        compiler_params=_sc_params,
    )
    return k(es_tbl, ed_tbl, src, dst)


def _scale_rows(rows, exch, head):
    @plsc.parallel_loop(0, _CH, unroll=4)
    def _edge(e):
        exv = exch[e, :][head]
        for cc in range(DP // L):
            v = rows[e, pl.ds(cc * L, L)]
            rows[e, pl.ds(cc * L, L)] = v * exv


def _sc_pass2_head(head, sid, hh_hbm, ex_hbm, out_hbm, src_all, dst2d,
                   exch0, exch1, rows0, rows1, zb, acc_sh, gs0, gs1, ss0, ss1):
    def _z(i, _):
        for cc in range(DP // L):
            zb[i, pl.ds(cc * L, L)] = jnp.zeros((L,), jnp.float32)
        return 0
    lax.fori_loop(0, 32, _z, 0)
    for j in range(_RPT // 32):
        pltpu.sync_copy(zb, acc_sh.at[pl.ds(sid * _RPT + j * 32, 32)])
    plsc.subcore_barrier()

    tbl = hh_hbm.at[head]
    base_t = sid * _EPT2

    def _gather(k, rows, exch, sem):
        pltpu.async_copy(ex_hbm.at[pl.ds(base_t + k * _CH, _CH)], exch, sem)
        pltpu.async_copy(tbl.at[src_all.at[pl.ds(k * _CH, _CH)]], rows, sem)

    def _gwait(k, rows, exch, sem):
        pltpu.make_async_copy(ex_hbm.at[pl.ds(base_t + k * _CH, _CH)],
                              exch, sem).wait()
        pltpu.make_async_copy(tbl.at[src_all.at[pl.ds(k * _CH, _CH)]],
                              rows, sem).wait()

    def _scat(k, rows, sem):
        pltpu.async_copy(rows, acc_sh.at[dst2d.at[k]], sem, add=True)

    def _swait(k, rows, sem):
        pltpu.make_async_copy(rows, acc_sh.at[dst2d.at[k]], sem).wait()

    _gather(0, rows0, exch0, gs0)

    def _pair(j, _):
        k0 = 2 * j
        _gwait(k0, rows0, exch0, gs0)

        @pl.when(j > 0)
        def _():
            _swait(k0, rows1, ss1)
        _gather(k0 + 1, rows1, exch1, gs1)
        _scale_rows(rows0, exch0, head)
        _scat(k0, rows0, ss0)
        _gwait(k0 + 1, rows1, exch1, gs1)
        _swait(k0, rows0, ss0)

        @pl.when(j < _NCH2 // 2 - 1)
        def _():
            _gather(k0 + 2, rows0, exch0, gs0)
        _scale_rows(rows1, exch1, head)
        _scat(k0 + 1, rows1, ss1)
        return 0

    lax.fori_loop(0, _NCH2 // 2, _pair, 0)
    _swait(_NCH2 - 1, rows1, ss1)
    plsc.subcore_barrier()
    pltpu.sync_copy(acc_sh.at[pl.ds(sid * _RPT, _RPT)],
                    out_hbm.at[head, pl.ds(sid * _RPT, _RPT)])
    plsc.subcore_barrier()


def _sc_pass2_body(hh_hbm, src_hbm, dst_hbm, ex_hbm, out_hbm,
                   src_all, dst2d, exch0, exch1, rows0, rows1, zb, acc_sh,
                   gs0, gs1, ss0, ss1):
    c = lax.axis_index("c")
    sid = lax.axis_index("s")
    base_t = sid * _EPT2
    pltpu.sync_copy(src_hbm.at[pl.ds(base_t, _EPT2)], src_all)
    pltpu.sync_copy(dst_hbm.at[pl.ds(sid * _NCH2, _NCH2)], dst2d)
    for core in range(NC):
        @pl.when(c == core)
        def _():
            for sub in range(2):
                _sc_pass2_head(2 * core + sub, sid, hh_hbm, ex_hbm, out_hbm,
                               src_all, dst2d, exch0, exch1, rows0, rows1,
                               zb, acc_sh, gs0, gs1, ss0, ss1)


def _sc_pass2(h_heads, src, dst2d, ex):
    k = pl.kernel(
        _sc_pass2_body,
        out_type=jax.ShapeDtypeStruct((H, N, DP), jnp.float32),
        mesh=_sc_mesh,
        scratch_types=[
            pltpu.VMEM((_EPT2,), jnp.int32),
            pltpu.VMEM((_NCH2, _CH), jnp.int32),
            pltpu.VMEM((_CH, 16), jnp.float32),
            pltpu.VMEM((_CH, 16), jnp.float32),
            pltpu.VMEM((_CH, DP), jnp.float32),
            pltpu.VMEM((_CH, DP), jnp.float32),
            pltpu.VMEM((32, DP), jnp.float32),
            pltpu.VMEM_SHARED((N, DP), jnp.float32),
            pltpu.SemaphoreType.DMA,
            pltpu.SemaphoreType.DMA,
            pltpu.SemaphoreType.DMA,
            pltpu.SemaphoreType.DMA,
        ],
        compiler_params=_sc_params,
    )
    return k(h_heads, src, dst2d, ex)


def _fold_att(W, a):
    S = jnp.zeros((H * HID, 16), jnp.float32)
    for h in range(H):
        S = S.at[h * HID:(h + 1) * HID, h].set(a[h])
    return W @ S


def kernel(feat_P, feat_A, cent_obs, edge_index, batch_size,
           W_P1, W_A1, a_src1, a_dst1, W_s1_self, W_s1_node,
           W_P2, W_A2, a_src2, a_dst2, W_s2, W_ns2):
    src = edge_index[0].astype(jnp.int32)
    dst = edge_index[1].astype(jnp.int32)
    dst2d = dst.reshape(E // _CH, _CH)
    feat = jnp.concatenate([feat_P, feat_A], axis=0)

    W1s = jnp.stack([W_P1, W_A1])
    Wa1 = jnp.stack([
        jnp.stack([_fold_att(W_P1, a_src1), _fold_att(W_P1, a_dst1)]),
        jnp.stack([_fold_att(W_A1, a_src1), _fold_att(W_A1, a_dst1)])])
    W2s = jnp.stack([W_P2, W_A2])
    Wa2 = jnp.stack([
        jnp.stack([_fold_att(W_P2, a_src2), _fold_att(W_P2, a_dst2)]),
        jnp.stack([_fold_att(W_A2, a_src2), _fold_att(W_A2, a_dst2)])])

    h1_heads, es1, ed1 = _tc_proj(feat, W1s, Wa1)
    ex1 = _sc_pass1(es1, ed1, src, dst)
    out1 = _sc_pass2(h1_heads, src, dst2d, ex1)

    h1sum, h2_heads, es2, ed2 = _tc_norm_proj(out1, W2s, Wa2)
    ex2 = _sc_pass1(es2, ed2, src, dst)
    out2 = _sc_pass2(h2_heads, src, dst2d, ex2)

    x, state = _tc_final(out2, h1sum, cent_obs, W_s1_self, W_s1_node,
                         W_s2, W_ns2)
    return (x, state)


# speedup vs baseline: 1.0071x; 1.0002x over previous
import jax
import jax.numpy as jnp
from jax import lax
from jax.experimental import pallas as pl
from jax.experimental.pallas import tpu as pltpu
from jax.experimental.pallas import tpu_sc as plsc

B, NP, NA, DIN, HID, H = 256, 16, 16, 256, 128, 4
N = B * (NP + NA)
E = 65536
NC, NS, L = 2, 16, 16
NW = NC * NS
DP = 144

_HI = jax.lax.Precision.DEFAULT


_BM = 512


def _pad_cols(bm):
    return jnp.concatenate(
        [jnp.ones((bm, 1), jnp.float32), jnp.zeros((bm, DP - HID - 1),
                                                   jnp.float32)], axis=1)


def _proj_body(feat_ref, w_ref, wa_ref, hh_ref, es_ref, ed_ref):
    h = pl.program_id(1)
    f = feat_ref[...]
    blk = jnp.dot(f, w_ref[0], preferred_element_type=jnp.float32,
                  precision=_HI)
    hh_ref[0] = jnp.concatenate([blk, _pad_cols(f.shape[0])], axis=1)

    @pl.when(h == 0)
    def _():
        es_ref[...] = jnp.dot(f, wa_ref[0, 0],
                              preferred_element_type=jnp.float32,
                              precision=_HI)
        ed_ref[...] = jnp.dot(f, wa_ref[0, 1],
                              preferred_element_type=jnp.float32,
                              precision=_HI)


def _tc_proj(feat, Ws, Wa):
    K = feat.shape[1]
    ni = N // _BM
    return pl.pallas_call(
        _proj_body,
        grid=(ni, H),
        in_specs=[
            pl.BlockSpec((_BM, K), lambda i, h: (i, 0)),
            pl.BlockSpec((1, K, HID), lambda i, h: (i // (ni // 2), 0, h)),
            pl.BlockSpec((1, 2, K, 16), lambda i, h: (i // (ni // 2), 0, 0, 0)),
        ],
        out_specs=[
            pl.BlockSpec((1, _BM, DP), lambda i, h: (h, i, 0)),
            pl.BlockSpec((_BM, 16), lambda i, h: (i, 0)),
            pl.BlockSpec((_BM, 16), lambda i, h: (i, 0)),
        ],
        out_shape=[
            jax.ShapeDtypeStruct((H, N, DP), jnp.float32),
            jax.ShapeDtypeStruct((N, 16), jnp.float32),
            jax.ShapeDtypeStruct((N, 16), jnp.float32),
        ],
    )(feat, Ws, Wa)


def _elu(x):
    return jnp.where(x > 0, x, jnp.exp(jnp.minimum(x, 0.0)) - 1.0)


def _h1_from(out_ref):
    cols = []
    for h in range(H):
        s = out_ref[h, :, HID:HID + 1] + 1e-9
        cols.append(_elu(out_ref[h, :, :HID] / s))
    return jnp.concatenate(cols, axis=1)


def _norm_proj_body(out_ref, w_ref, wa_ref, h1sum_ref, hh_ref, es_ref,
                    ed_ref):
    i = pl.program_id(0)
    h = pl.program_id(1)
    h1 = _h1_from(out_ref)
    blk = jnp.dot(h1, w_ref[0], preferred_element_type=jnp.float32,
                  precision=_HI)
    hh_ref[0] = jnp.concatenate([blk, _pad_cols(h1.shape[0])], axis=1)

    @pl.when(h == 0)
    def _():
        es_ref[...] = jnp.dot(h1, wa_ref[0, 0],
                              preferred_element_type=jnp.float32,
                              precision=_HI)
        ed_ref[...] = jnp.dot(h1, wa_ref[0, 1],
                              preferred_element_type=jnp.float32,
                              precision=_HI)

        @pl.when(i == 0)
        def _():
            h1sum_ref[...] = jnp.zeros((B, H * HID), jnp.float32)
        ng = _BM // NP
        g0 = (i % (B // ng)) * ng
        h1sum_ref[pl.ds(g0, ng), :] = (
            h1sum_ref[pl.ds(g0, ng), :]
            + h1.reshape(ng, NP, H * HID).sum(axis=1))


def _tc_norm_proj(out_unnorm, Ws, Wa):
    ni = N // _BM
    return pl.pallas_call(
        _norm_proj_body,
        grid=(ni, H),
        in_specs=[
            pl.BlockSpec((H, _BM, DP), lambda i, h: (0, i, 0)),
            pl.BlockSpec((1, H * HID, HID),
                         lambda i, h: (i // (ni // 2), 0, h)),
            pl.BlockSpec((1, 2, H * HID, 16),
                         lambda i, h: (i // (ni // 2), 0, 0, 0)),
        ],
        out_specs=[
            pl.BlockSpec((B, H * HID), lambda i, h: (0, 0)),
            pl.BlockSpec((1, _BM, DP), lambda i, h: (h, i, 0)),
            pl.BlockSpec((_BM, 16), lambda i, h: (i, 0)),
            pl.BlockSpec((_BM, 16), lambda i, h: (i, 0)),
        ],
        out_shape=[
            jax.ShapeDtypeStruct((B, H * HID), jnp.float32),
            jax.ShapeDtypeStruct((H, N, DP), jnp.float32),
            jax.ShapeDtypeStruct((N, 16), jnp.float32),
            jax.ShapeDtypeStruct((N, 16), jnp.float32),
        ],
    )(out_unnorm, Ws, Wa)


def _final_body(out_ref, h1sum_ref, obs_ref, wself_ref, wnode_ref,
                ws2_ref, wns2_ref, x_ref, state_ref):
    st1 = _elu(jnp.dot(obs_ref[...], wself_ref[...],
                       preferred_element_type=jnp.float32, precision=_HI)
               + jnp.dot(h1sum_ref[...] * (1.0 / (NP + NA)), wnode_ref[...],
                         preferred_element_type=jnp.float32, precision=_HI))
    q = jnp.zeros((N, HID), jnp.float32)
    for h in range(H):
        s = out_ref[h, :, HID:HID + 1] + 1e-9
        q = q + out_ref[h, :, :HID] / s
    q = q * (1.0 / H)
    x = jnp.concatenate([q[:B * NP].reshape(B, NP, HID),
                         q[B * NP:].reshape(B, NA, HID)], axis=1)
    x_ref[...] = x
    nm2 = x.sum(axis=1) * (1.0 / (NP + NA))
    state_ref[...] = (
        jnp.dot(st1, ws2_ref[...],
                preferred_element_type=jnp.float32, precision=_HI)
        + jnp.dot(nm2, wns2_ref[...],
                  preferred_element_type=jnp.float32, precision=_HI))


def _tc_final(out_unnorm, h1sum, cent_obs, W_self, W_node, W_s2, W_ns2):
    return pl.pallas_call(
        _final_body,
        out_shape=[
            jax.ShapeDtypeStruct((B, NP + NA, HID), jnp.float32),
            jax.ShapeDtypeStruct((B, W_s2.shape[1]), jnp.float32),
        ],
    )(out_unnorm, h1sum, cent_obs, W_self, W_node, W_s2, W_ns2)


_CH = 128
_EPW = E // NW
_NCH1 = _EPW // _CH
_RPT = N // NS
_EPT2 = E // NS
_NCH2 = _EPT2 // _CH

_sc_mesh = plsc.VectorSubcoreMesh(core_axis_name="c", subcore_axis_name="s")
_sc_params = pltpu.CompilerParams(use_tc_tiling_on_sc=False)


def _sc_pass1_body(es_hbm, ed_hbm, src_hbm, dst_hbm, ex_hbm,
                   src_all, dst_all, srows0, drows0, srows1, drows1,
                   exfull, gs0, gs1):
    c = lax.axis_index("c")
    sid = lax.axis_index("s")
    wid = sid * NC + c
    base_w = wid * _EPW
    pltpu.sync_copy(src_hbm.at[pl.ds(base_w, _EPW)], src_all)
    pltpu.sync_copy(dst_hbm.at[pl.ds(base_w, _EPW)], dst_all)

    def _gather(k, srows, drows, sem):
        pltpu.async_copy(es_hbm.at[src_all.at[pl.ds(k * _CH, _CH)]],
                         srows, sem)
        pltpu.async_copy(ed_hbm.at[dst_all.at[pl.ds(k * _CH, _CH)]],
                         drows, sem)

    def _gwait(k, srows, drows, sem):
        pltpu.make_async_copy(es_hbm.at[src_all.at[pl.ds(k * _CH, _CH)]],
                              srows, sem).wait()
        pltpu.make_async_copy(ed_hbm.at[dst_all.at[pl.ds(k * _CH, _CH)]],
                              drows, sem).wait()

    def _compute(k, srows, drows):
        @plsc.parallel_loop(0, _CH, unroll=4)
        def _edge(e):
            z = srows[e, :] + drows[e, :]
            z = jnp.where(z >= 0.0, z, z * jnp.float32(0.2))
            exfull[k * _CH + e, :] = jnp.exp(z)

    _gather(0, srows0, drows0, gs0)

    def _pair(j, _):
        k0 = 2 * j
        _gwait(k0, srows0, drows0, gs0)
        _gather(k0 + 1, srows1, drows1, gs1)
        _compute(k0, srows0, drows0)
        _gwait(k0 + 1, srows1, drows1, gs1)

        @pl.when(j < _NCH1 // 2 - 1)
        def _():
            _gather(k0 + 2, srows0, drows0, gs0)
        _compute(k0 + 1, srows1, drows1)
        return 0

    lax.fori_loop(0, _NCH1 // 2, _pair, 0)
    pltpu.sync_copy(exfull, ex_hbm.at[pl.ds(base_w, _EPW)])


def _sc_pass1(es_tbl, ed_tbl, src, dst):
    k = pl.kernel(
        _sc_pass1_body,
        out_type=jax.ShapeDtypeStruct((E, 16), jnp.float32),
        mesh=_sc_mesh,
        scratch_types=[
            pltpu.VMEM((_EPW,), jnp.int32),
            pltpu.VMEM((_EPW,), jnp.int32),
            pltpu.VMEM((_CH, 16), jnp.float32),
            pltpu.VMEM((_CH, 16), jnp.float32),
            pltpu.VMEM((_CH, 16), jnp.float32),
            pltpu.VMEM((_CH, 16), jnp.float32),
            pltpu.VMEM((_EPW, 16), jnp.float32),
            pltpu.SemaphoreType.DMA,
            pltpu.SemaphoreType.DMA,
        ],
        compiler_params=_sc_params,
    )
    return k(es_tbl, ed_tbl, src, dst)


def _scale_rows(rows, exch, head):
    @plsc.parallel_loop(0, _CH, unroll=4)
    def _edge(e):
        exv = exch[e, :][head]
        for cc in range(DP // L):
            v = rows[e, pl.ds(cc * L, L)]
            rows[e, pl.ds(cc * L, L)] = v * exv


def _sc_pass2_head(head, sid, hh_hbm, ex_hbm, out_hbm, src_all, dst2d,
                   exch0, exch1, rows0, rows1, zb, acc_sh, gs0, gs1, ss0, ss1):
    def _z(i, _):
        for cc in range(DP // L):
            zb[i, pl.ds(cc * L, L)] = jnp.zeros((L,), jnp.float32)
        return 0
    lax.fori_loop(0, 32, _z, 0)
    for j in range(_RPT // 32):
        pltpu.sync_copy(zb, acc_sh.at[pl.ds(sid * _RPT + j * 32, 32)])
    plsc.subcore_barrier()

    tbl = hh_hbm.at[head]
    base_t = sid * _EPT2

    def _gather(k, rows, exch, sem):
        pltpu.async_copy(ex_hbm.at[pl.ds(base_t + k * _CH, _CH)], exch, sem)
        pltpu.async_copy(tbl.at[src_all.at[pl.ds(k * _CH, _CH)]], rows, sem)

    def _gwait(k, rows, exch, sem):
        pltpu.make_async_copy(ex_hbm.at[pl.ds(base_t + k * _CH, _CH)],
                              exch, sem).wait()
        pltpu.make_async_copy(tbl.at[src_all.at[pl.ds(k * _CH, _CH)]],
                              rows, sem).wait()

    def _scat(k, rows, sem):
        pltpu.async_copy(rows, acc_sh.at[dst2d.at[k]], sem, add=True)

    def _swait(k, rows, sem):
        pltpu.make_async_copy(rows, acc_sh.at[dst2d.at[k]], sem).wait()

    _gather(0, rows0, exch0, gs0)

    def _pair(j, _):
        k0 = 2 * j
        _gwait(k0, rows0, exch0, gs0)

        @pl.when(j > 0)
        def _():
            _swait(k0, rows1, ss1)
        _gather(k0 + 1, rows1, exch1, gs1)
        _scale_rows(rows0, exch0, head)
        _scat(k0, rows0, ss0)
        _gwait(k0 + 1, rows1, exch1, gs1)
        _swait(k0, rows0, ss0)

        @pl.when(j < _NCH2 // 2 - 1)
        def _():
            _gather(k0 + 2, rows0, exch0, gs0)
        _scale_rows(rows1, exch1, head)
        _scat(k0 + 1, rows1, ss1)
        return 0

    lax.fori_loop(0, _NCH2 // 2, _pair, 0)
    _swait(_NCH2 - 1, rows1, ss1)
    plsc.subcore_barrier()
    pltpu.sync_copy(acc_sh.at[pl.ds(sid * _RPT, _RPT)],
                    out_hbm.at[head, pl.ds(sid * _RPT, _RPT)])
    plsc.subcore_barrier()


def _sc_pass2_body(hh_hbm, src_hbm, dst_hbm, ex_hbm, out_hbm,
                   src_all, dst2d, exch0, exch1, rows0, rows1, zb, acc_sh,
                   gs0, gs1, ss0, ss1):
    c = lax.axis_index("c")
    sid = lax.axis_index("s")
    base_t = sid * _EPT2
    pltpu.sync_copy(src_hbm.at[pl.ds(base_t, _EPT2)], src_all)
    pltpu.sync_copy(dst_hbm.at[pl.ds(sid * _NCH2, _NCH2)], dst2d)
    for core in range(NC):
        @pl.when(c == core)
        def _():
            for sub in range(2):
                _sc_pass2_head(2 * core + sub, sid, hh_hbm, ex_hbm, out_hbm,
                               src_all, dst2d, exch0, exch1, rows0, rows1,
                               zb, acc_sh, gs0, gs1, ss0, ss1)


def _sc_pass2(h_heads, src, dst2d, ex):
    k = pl.kernel(
        _sc_pass2_body,
        out_type=jax.ShapeDtypeStruct((H, N, DP), jnp.float32),
        mesh=_sc_mesh,
        scratch_types=[
            pltpu.VMEM((_EPT2,), jnp.int32),
            pltpu.VMEM((_NCH2, _CH), jnp.int32),
            pltpu.VMEM((_CH, 16), jnp.float32),
            pltpu.VMEM((_CH, 16), jnp.float32),
            pltpu.VMEM((_CH, DP), jnp.float32),
            pltpu.VMEM((_CH, DP), jnp.float32),
            pltpu.VMEM((32, DP), jnp.float32),
            pltpu.VMEM_SHARED((N, DP), jnp.float32),
            pltpu.SemaphoreType.DMA,
            pltpu.SemaphoreType.DMA,
            pltpu.SemaphoreType.DMA,
            pltpu.SemaphoreType.DMA,
        ],
        compiler_params=_sc_params,
    )
    return k(h_heads, src, dst2d, ex)


def _fold_att(W, a):
    S = jnp.zeros((H * HID, 16), jnp.float32)
    for h in range(H):
        S = S.at[h * HID:(h + 1) * HID, h].set(a[h])
    return W @ S


def kernel(feat_P, feat_A, cent_obs, edge_index, batch_size,
           W_P1, W_A1, a_src1, a_dst1, W_s1_self, W_s1_node,
           W_P2, W_A2, a_src2, a_dst2, W_s2, W_ns2):
    src = edge_index[0].astype(jnp.int32)
    dst = edge_index[1].astype(jnp.int32)
    dst2d = dst.reshape(E // _CH, _CH)
    feat = jnp.concatenate([feat_P, feat_A], axis=0)

    W1s = jnp.stack([W_P1, W_A1])
    Wa1 = jnp.stack([
        jnp.stack([_fold_att(W_P1, a_src1), _fold_att(W_P1, a_dst1)]),
        jnp.stack([_fold_att(W_A1, a_src1), _fold_att(W_A1, a_dst1)])])
    W2s = jnp.stack([W_P2, W_A2])
    Wa2 = jnp.stack([
        jnp.stack([_fold_att(W_P2, a_src2), _fold_att(W_P2, a_dst2)]),
        jnp.stack([_fold_att(W_A2, a_src2), _fold_att(W_A2, a_dst2)])])

    h1_heads, es1, ed1 = _tc_proj(feat, W1s, Wa1)
    ex1 = _sc_pass1(es1, ed1, src, dst)
    out1 = _sc_pass2(h1_heads, src, dst2d, ex1)

    h1sum, h2_heads, es2, ed2 = _tc_norm_proj(out1, W2s, Wa2)
    ex2 = _sc_pass1(es2, ed2, src, dst)
    out2 = _sc_pass2(h2_heads, src, dst2d, ex2)

    x, state = _tc_final(out2, h1sum, cent_obs, W_s1_self, W_s1_node,
                         W_s2, W_ns2)
    return (x, state)
